# baseline (speedup 1.0000x reference)
import functools

import jax
import jax.numpy as jnp
from jax import lax
from jax.experimental import pallas as pl
from jax.experimental.pallas import tpu as pltpu
from jax.experimental.pallas import tpu_sc as plsc

_B, _S, _D = 64, 499, 256
_SP = 512
_NW = 32
_RPW = (_B * _SP) // _NW
_CH = 128
_NCH = _RPW // _CH


def _sc_gather_body(emb_p_h, emb_q_h, idx_h, o0, o1, o2, o3,
                    idx_v, rows0, rows1, gs0, gs1, ws0, ws1):
    wid = lax.axis_index("s") * 2 + lax.axis_index("c")
    base = wid * _RPW
    tabs = (emb_p_h, emb_q_h, emb_q_h, emb_p_h)
    outs = (o0, o1, o2, o3)
    for t in range(4):
        pltpu.sync_copy(idx_h.at[t, wid], idx_v.at[t])
    bufs = (rows0, rows1)
    gsem = (gs0, gs1)
    wsem = (ws0, ws1)
    seq = [(t, c) for t in range(4) for c in range(_NCH)]
    n = len(seq)
    gcp = [None, None]
    wcp = [None, None]
    gcp[0] = pltpu.async_copy(tabs[0].at[idx_v.at[0, 0]], bufs[0], gs0)
    for k in range(n):
        s = k % 2
        t, c = seq[k]
        gcp[s].wait()
        if k + 1 < n:
            s2 = (k + 1) % 2
            t2, c2 = seq[k + 1]
            if wcp[s2] is not None:
                wcp[s2].wait()
            gcp[s2] = pltpu.async_copy(tabs[t2].at[idx_v.at[t2, c2]],
                                       bufs[s2], gsem[s2])
        wcp[s] = pltpu.async_copy(
            bufs[s], outs[t].at[pl.ds(base + c * _CH, _CH)], wsem[s])
    wcp[0].wait()
    wcp[1].wait()


def _sc_gather(emb_p, emb_q, idx4):
    row = jax.ShapeDtypeStruct((_B * _SP, _D), jnp.float32)
    mesh = plsc.VectorSubcoreMesh(core_axis_name="c", subcore_axis_name="s")
    f = functools.partial(
        pl.kernel,
        out_type=[row, row, row, row],
        mesh=mesh,
        scratch_types=[
            pltpu.VMEM((4, _NCH, _CH), jnp.int32),
            pltpu.VMEM((_CH, _D), jnp.float32),
            pltpu.VMEM((_CH, _D), jnp.float32),
            pltpu.SemaphoreType.DMA,
            pltpu.SemaphoreType.DMA,
            pltpu.SemaphoreType.DMA,
            pltpu.SemaphoreType.DMA,
        ],
    )(_sc_gather_body)
    return f(emb_p, emb_q, idx4)


def _prelude_body(tl_ref,
                  pe_ref, qe_ref, aff_ref, r_ref,
                  Wa_ref, baff_ref, W1T_ref, As1_ref, Ad1_ref, E8_ref, b1_ref,
                  W2T_ref, as2_ref, ad2_ref, b2_ref,
                  WihT_ref, bih_ref, embr_ref, embaff_ref,
                  zx_ref):
    b = pl.program_id(0)
    f32 = jnp.float32
    bf16 = jnp.bfloat16
    dot = lambda a, w: jnp.dot(a, w, preferred_element_type=f32)

    n = tl_ref[b] - 1
    jcol = lax.broadcasted_iota(jnp.int32, (_SP, 1), 0)
    mp = (jcol >= 1) & (jcol < n)
    mn = (jcol + 1) < n

    pe = pe_ref[0].astype(bf16)
    qe = qe_ref[0].astype(bf16)

    a_col = aff_ref[0]
    oh = (a_col == lax.broadcasted_iota(jnp.int32, (_SP, 16), 1)).astype(bf16)
    affproj = dot(embaff_ref[...], Wa_ref[_D:]).astype(bf16)
    x = (dot(pe, Wa_ref[:_D]) + dot(oh, affproj) + baff_ref[...]).astype(bf16)

    neg = f32(-1e30)
    lrelu = lambda v: jnp.where(v >= 0, v, 0.2 * v)

    def stencil(hmat, a_s, a_d, nlanes):
        zrow = jnp.zeros((1, nlanes), f32)
        a_sp = jnp.concatenate([jnp.zeros((1, a_s.shape[1]), f32), a_s[:-1]], 0)
        a_sn = jnp.concatenate([a_s[1:], jnp.zeros((1, a_s.shape[1]), f32)], 0)
        al_s = lrelu(a_s + a_d)
        al_p = lrelu(a_sp + a_d)
        al_n = lrelu(a_sn + a_d)
        amax = jnp.maximum(al_s, jnp.maximum(jnp.where(mp, al_p, neg),
                                             jnp.where(mn, al_n, neg)))
        e_s = jnp.exp(al_s - amax)
        e_p = jnp.where(mp, jnp.exp(al_p - amax), 0.0)
        e_n = jnp.where(mn, jnp.exp(al_n - amax), 0.0)
        den = e_s + e_p + e_n + 1e-16
        h_p = jnp.concatenate([zrow, hmat[:-1]], 0)
        h_n = jnp.concatenate([hmat[1:], zrow], 0)
        return (e_s / den), (e_p / den), (e_n / den), h_p, h_n

    h1 = dot(x, W1T_ref[...])
    h1b = h1.astype(bf16)
    a_s1 = dot(h1b, As1_ref[...])
    a_d1 = dot(h1b, Ad1_ref[...])
    cs, cp, cn, h1p, h1n = stencil(h1, a_s1, a_d1, 8 * 128)
    g1 = (dot(cs.astype(bf16), E8_ref[...]) * h1
          + dot(cp.astype(bf16), E8_ref[...]) * h1p
          + dot(cn.astype(bf16), E8_ref[...]) * h1n + b1_ref[...])
    x1 = jnp.where(g1 > 0, g1, jnp.exp(jnp.minimum(g1, 0.0)) - 1.0)
    x1 = x1.astype(bf16)

    h2 = dot(x1, W2T_ref[...])
    h2b = h2.astype(bf16)
    a_s2 = dot(h2b, as2_ref[...])
    a_d2 = dot(h2b, ad2_ref[...])
    cs2, cp2, cn2, h2p, h2n = stencil(h2, a_s2, a_d2, _D)
    aff_out = cs2 * h2 + cp2 * h2p + cn2 * h2n + b2_ref[...]

    z = (dot(pe, WihT_ref[:_D]) + dot(qe, WihT_ref[_D:2 * _D])
         + dot(aff_out.astype(bf16), WihT_ref[3 * _D:]) + bih_ref[...])
    zr = dot(embr_ref[...], WihT_ref[2 * _D:3 * _D])
    rf = r_ref[0].astype(f32)
    z = z + zr[0:1] + rf * (zr[1:2] - zr[0:1])
    for k in range(4):
        zx_ref[0, :, k, :] = z[:, k * _D:(k + 1) * _D].astype(bf16)


_VSPEC = lambda shp, imap: pl.BlockSpec(shp, imap)
_FULL = lambda shp: pl.BlockSpec(shp, lambda b: (0,) * len(shp))
_ROW = lambda shp: pl.BlockSpec(shp, lambda b: (b,) + (0,) * (len(shp) - 1))

_PRELUDE_KW = dict(
    grid=(_B,),
    in_specs=[
        pl.BlockSpec(memory_space=pltpu.SMEM),
        _ROW((1, _SP, _D)),
        _ROW((1, _SP, _D)),
        _ROW((1, _SP, 1)),
        _ROW((1, _SP, 1)),
        _FULL((2 * _D, _D)),
        _FULL((1, _D)),
        _FULL((_D, 8 * 128)),
        _FULL((8 * 128, 8)),
        _FULL((8 * 128, 8)),
        _FULL((8, 8 * 128)),
        _FULL((1, 8 * 128)),
        _FULL((8 * 128, _D)),
        _FULL((_D, 1)),
        _FULL((_D, 1)),
        _FULL((1, _D)),
        _FULL((4 * _D, 4 * _D)),
        _FULL((1, 4 * _D)),
        _FULL((2, _D)),
        _FULL((16, _D)),
    ],
    out_specs=pl.BlockSpec((1, _SP, 4, _D), lambda b: (b, 0, 0, 0)),
    out_shape=jax.ShapeDtypeStruct((_B, _SP, 4, _D), jnp.bfloat16),
    compiler_params=pltpu.CompilerParams(dimension_semantics=("arbitrary",)),
)


def _lstm_body(z_ref, qn_ref, pn_ref, bout_ref, whh_ref, bhh_ref, wh_ref,
               wqn_ref, wpn_ref, out_ref, h_ref, c_ref):
    t = pl.program_id(0)
    bf16 = jnp.bfloat16

    @pl.when(t == 0)
    def _init():
        h_ref[...] = jnp.zeros((_B, _D), jnp.float32)
        c_ref[...] = jnp.zeros((_B, _D), jnp.float32)

    h = h_ref[...]
    c = c_ref[...]
    zh = jnp.dot(h, whh_ref[...], preferred_element_type=jnp.float32) \
        + bhh_ref[...]
    zx = z_ref[:, 0].astype(jnp.float32)
    zi = zx[:, 0] + zh[:, 0:_D]
    zf = zx[:, 1] + zh[:, _D:2 * _D]
    zg = zx[:, 2] + zh[:, 2 * _D:3 * _D]
    zo = zx[:, 3] + zh[:, 3 * _D:]
    sig = lambda v: 1.0 / (1.0 + jnp.exp(-v))
    c_new = sig(zf) * c + sig(zi) * jnp.tanh(zg)
    h_new = sig(zo) * jnp.tanh(c_new)
    h_ref[...] = h_new
    c_ref[...] = c_new
    hp = jnp.dot(h_new, wh_ref[...], preferred_element_type=jnp.float32)
    s_t = jnp.dot(qn_ref[:, 0, 0].astype(bf16), wqn_ref[...],
                  preferred_element_type=jnp.float32) \
        + jnp.dot(pn_ref[:, 0, 0].astype(bf16), wpn_ref[...],
                  preferred_element_type=jnp.float32) + bout_ref[0]
    out_ref[...] = sig(hp + s_t).reshape(1, _B, 1)


_LSTM_KW = dict(
    grid=(_S,),
    in_specs=[
        pl.BlockSpec((_B, 1, 4, _D), lambda t: (0, t, 0, 0)),
        pl.BlockSpec((_B, 1, 1, _D), lambda t: (0, t, 0, 0)),
        pl.BlockSpec((_B, 1, 1, _D), lambda t: (0, t, 0, 0)),
        pl.BlockSpec(memory_space=pltpu.SMEM),
        _FULL((_D, 4 * _D)),
        _FULL((1, 4 * _D)),
        _FULL((_D, 1)),
        _FULL((_D, 1)),
        _FULL((_D, 1)),
    ],
    out_specs=pl.BlockSpec((1, _B, 1), lambda t: (t, 0, 0)),
    out_shape=jax.ShapeDtypeStruct((_S, _B, 1), jnp.float32),
    scratch_shapes=[
        pltpu.VMEM((_B, _D), jnp.float32),
        pltpu.VMEM((_B, _D), jnp.float32),
    ],
    compiler_params=pltpu.CompilerParams(dimension_semantics=("arbitrary",)),
)


def kernel(true_length, p, q, r, aff, q_next, p_next, emb_p, emb_q, emb_r,
           emb_aff, W_aff, b_aff, W1, att_src1, att_dst1, b1, W2, att_src2,
           att_dst2, b2, W_ih, W_hh, b_ih, b_hh, W_out, b_out):
    f32 = jnp.float32
    i32 = jnp.int32
    pad = lambda a: jnp.pad(a.astype(i32), ((0, 0), (0, _SP - _S)))

    idx4 = jnp.stack([pad(p), pad(q), pad(q_next), pad(p_next)]) \
        .reshape(4, _NW, _NCH, _CH)
    pe, qe, qn, pn = [g.reshape(_B, _SP, _D)
                      for g in _sc_gather(emb_p.astype(f32), emb_q.astype(f32),
                                          idx4)]

    aff4 = pad(aff)[..., None]
    r4 = pad(r)[..., None]

    bf16 = jnp.bfloat16
    Wa = W_aff.T.astype(bf16)
    As1 = (att_src1[:, :, None] * jnp.eye(8, dtype=f32)[:, None, :]) \
        .reshape(8 * 128, 8).astype(bf16)
    Ad1 = (att_dst1[:, :, None] * jnp.eye(8, dtype=f32)[:, None, :]) \
        .reshape(8 * 128, 8).astype(bf16)
    E8 = jnp.kron(jnp.eye(8, dtype=f32), jnp.ones((1, 128), f32)).astype(bf16)
    embaff16 = jnp.pad(emb_aff, ((0, 5), (0, 0))).astype(bf16)
    wqn = W_out[0, _D:2 * _D][:, None].astype(bf16)
    wpn = W_out[0, 2 * _D:][:, None].astype(bf16)
    wh = W_out[0, :_D][:, None]

    zx = pl.pallas_call(_prelude_body, **_PRELUDE_KW)(
        true_length.astype(i32),
        pe, qe, aff4, r4,
        Wa, b_aff[None], W1.T.astype(bf16), As1, Ad1, E8, b1[None],
        W2.T.astype(bf16), att_src2.T.astype(bf16), att_dst2.T.astype(bf16),
        b2[None],
        W_ih.T.astype(bf16), b_ih[None], emb_r.astype(bf16), embaff16)

    out_tb = pl.pallas_call(_lstm_body, **_LSTM_KW)(
        zx, qn.reshape(_B, _SP, 1, _D), pn.reshape(_B, _SP, 1, _D),
        b_out.astype(f32), W_hh.T, b_hh[None], wh, wqn, wpn)

    return jnp.transpose(out_tb, (1, 0, 2))


# speedup vs baseline: 1.2310x; 1.2310x over previous
import functools

import jax
import jax.numpy as jnp
from jax import lax
from jax.experimental import pallas as pl
from jax.experimental.pallas import tpu as pltpu
from jax.experimental.pallas import tpu_sc as plsc

_B, _S, _D = 64, 499, 256
_SP = 512
_NW = 32
_RPW = (_B * _SP) // _NW
_CH = 128
_NCH = _RPW // _CH


def _sc_gather_body(emb_p_h, emb_q_h, idx_h, o0, o1, o2, o3,
                    idx_v, rows0, rows1, gs0, gs1, ws0, ws1):
    wid = lax.axis_index("s") * 2 + lax.axis_index("c")
    base = wid * _RPW
    tabs = (emb_p_h, emb_q_h, emb_q_h, emb_p_h)
    outs = (o0, o1, o2, o3)
    for t in range(4):
        pltpu.sync_copy(idx_h.at[t, wid], idx_v.at[t])
    bufs = (rows0, rows1)
    gsem = (gs0, gs1)
    wsem = (ws0, ws1)
    seq = [(t, c) for t in range(4) for c in range(_NCH)]
    n = len(seq)
    gcp = [None, None]
    wcp = [None, None]
    gcp[0] = pltpu.async_copy(tabs[0].at[idx_v.at[0, 0]], bufs[0], gs0)
    for k in range(n):
        s = k % 2
        t, c = seq[k]
        gcp[s].wait()
        if k + 1 < n:
            s2 = (k + 1) % 2
            t2, c2 = seq[k + 1]
            if wcp[s2] is not None:
                wcp[s2].wait()
            gcp[s2] = pltpu.async_copy(tabs[t2].at[idx_v.at[t2, c2]],
                                       bufs[s2], gsem[s2])
        wcp[s] = pltpu.async_copy(
            bufs[s], outs[t].at[pl.ds(base + c * _CH, _CH)], wsem[s])
    wcp[0].wait()
    wcp[1].wait()


def _sc_gather(emb_p, emb_q, idx4):
    row = jax.ShapeDtypeStruct((_B * _SP, _D), jnp.float32)
    mesh = plsc.VectorSubcoreMesh(core_axis_name="c", subcore_axis_name="s")
    f = functools.partial(
        pl.kernel,
        out_type=[row, row, row, row],
        mesh=mesh,
        scratch_types=[
            pltpu.VMEM((4, _NCH, _CH), jnp.int32),
            pltpu.VMEM((_CH, _D), jnp.float32),
            pltpu.VMEM((_CH, _D), jnp.float32),
            pltpu.SemaphoreType.DMA,
            pltpu.SemaphoreType.DMA,
            pltpu.SemaphoreType.DMA,
            pltpu.SemaphoreType.DMA,
        ],
    )(_sc_gather_body)
    return f(emb_p, emb_q, idx4)


def _prelude_body(tl_ref, bout_ref,
                  pe_ref, qe_ref, qn_ref, pn_ref, aff_ref, r_ref,
                  Wa_ref, baff_ref, W1T_ref, As1_ref, Ad1_ref, E8_ref, b1_ref,
                  W2T_ref, as2_ref, ad2_ref, b2_ref,
                  WihT_ref, bih_ref, embr_ref, embaff_ref, wqn_ref, wpn_ref,
                  zx_ref, scal_ref):
    b = pl.program_id(0)
    f32 = jnp.float32
    dot = lambda a, w: jnp.dot(a, w, preferred_element_type=f32)

    n = tl_ref[b] - 1
    jcol = lax.broadcasted_iota(jnp.int32, (_SP, 1), 0)
    mp = (jcol >= 1) & (jcol < n)
    mn = (jcol + 1) < n

    pe = pe_ref[0]
    qe = qe_ref[0]

    a_col = aff_ref[0]
    oh = (a_col == lax.broadcasted_iota(jnp.int32, (_SP, 16), 1)).astype(f32)
    affproj = dot(embaff_ref[...], Wa_ref[_D:])
    x = dot(pe, Wa_ref[:_D]) + dot(oh, affproj) + baff_ref[...]

    neg = f32(-1e30)
    lrelu = lambda v: jnp.where(v >= 0, v, 0.2 * v)

    def stencil(hmat, a_s, a_d, nlanes):
        zrow = jnp.zeros((1, nlanes), f32)
        a_sp = jnp.concatenate([jnp.zeros((1, a_s.shape[1]), f32), a_s[:-1]], 0)
        a_sn = jnp.concatenate([a_s[1:], jnp.zeros((1, a_s.shape[1]), f32)], 0)
        al_s = lrelu(a_s + a_d)
        al_p = lrelu(a_sp + a_d)
        al_n = lrelu(a_sn + a_d)
        amax = jnp.maximum(al_s, jnp.maximum(jnp.where(mp, al_p, neg),
                                             jnp.where(mn, al_n, neg)))
        e_s = jnp.exp(al_s - amax)
        e_p = jnp.where(mp, jnp.exp(al_p - amax), 0.0)
        e_n = jnp.where(mn, jnp.exp(al_n - amax), 0.0)
        den = e_s + e_p + e_n + 1e-16
        h_p = jnp.concatenate([zrow, hmat[:-1]], 0)
        h_n = jnp.concatenate([hmat[1:], zrow], 0)
        return (e_s / den), (e_p / den), (e_n / den), h_p, h_n

    h1 = dot(x, W1T_ref[...])
    a_s1 = dot(h1, As1_ref[...])
    a_d1 = dot(h1, Ad1_ref[...])
    cs, cp, cn, h1p, h1n = stencil(h1, a_s1, a_d1, 8 * 128)
    g1 = (dot(cs, E8_ref[...]) * h1 + dot(cp, E8_ref[...]) * h1p
          + dot(cn, E8_ref[...]) * h1n + b1_ref[...])
    x1 = jnp.where(g1 > 0, g1, jnp.exp(jnp.minimum(g1, 0.0)) - 1.0)

    h2 = dot(x1, W2T_ref[...])
    a_s2 = dot(h2, as2_ref[...])
    a_d2 = dot(h2, ad2_ref[...])
    cs2, cp2, cn2, h2p, h2n = stencil(h2, a_s2, a_d2, _D)
    aff_out = cs2 * h2 + cp2 * h2p + cn2 * h2n + b2_ref[...]

    z = (dot(pe, WihT_ref[:_D]) + dot(qe, WihT_ref[_D:2 * _D])
         + dot(aff_out, WihT_ref[3 * _D:]) + bih_ref[...])
    zr = dot(embr_ref[...], WihT_ref[2 * _D:3 * _D])
    rf = r_ref[0].astype(f32)
    z = z + zr[0:1] + rf * (zr[1:2] - zr[0:1])
    for k in range(4):
        zx_ref[0, :, k, :] = z[:, k * _D:(k + 1) * _D]

    scal_ref[0] = dot(qn_ref[0], wqn_ref[...]) + dot(pn_ref[0], wpn_ref[...]) \
        + bout_ref[0]


_FULL = lambda shp: pl.BlockSpec(shp, lambda b: (0,) * len(shp))
_ROW = lambda shp: pl.BlockSpec(shp, lambda b: (b,) + (0,) * (len(shp) - 1))

_PRELUDE_KW = dict(
    grid=(_B,),
    in_specs=[
        pl.BlockSpec(memory_space=pltpu.SMEM),
        pl.BlockSpec(memory_space=pltpu.SMEM),
        _ROW((1, _SP, _D)),
        _ROW((1, _SP, _D)),
        _ROW((1, _SP, _D)),
        _ROW((1, _SP, _D)),
        _ROW((1, _SP, 1)),
        _ROW((1, _SP, 1)),
        _FULL((2 * _D, _D)),
        _FULL((1, _D)),
        _FULL((_D, 8 * 128)),
        _FULL((8 * 128, 8)),
        _FULL((8 * 128, 8)),
        _FULL((8, 8 * 128)),
        _FULL((1, 8 * 128)),
        _FULL((8 * 128, _D)),
        _FULL((_D, 1)),
        _FULL((_D, 1)),
        _FULL((1, _D)),
        _FULL((4 * _D, 4 * _D)),
        _FULL((1, 4 * _D)),
        _FULL((2, _D)),
        _FULL((16, _D)),
        _FULL((_D, 1)),
        _FULL((_D, 1)),
    ],
    out_specs=[
        pl.BlockSpec((1, _SP, 4, _D), lambda b: (b, 0, 0, 0)),
        pl.BlockSpec((1, _SP, 1), lambda b: (b, 0, 0)),
    ],
    out_shape=[
        jax.ShapeDtypeStruct((_B, _SP, 4, _D), jnp.float32),
        jax.ShapeDtypeStruct((_B, _SP, 1), jnp.float32),
    ],
    compiler_params=pltpu.CompilerParams(dimension_semantics=("arbitrary",)),
)


def _lstm_body(z_ref, scal_ref, whh_ref, bhh_ref, wh_ref, out_ref, h_ref, c_ref):
    t = pl.program_id(0)

    @pl.when(t == 0)
    def _init():
        h_ref[...] = jnp.zeros((_B, _D), jnp.float32)
        c_ref[...] = jnp.zeros((_B, _D), jnp.float32)

    h = h_ref[...]
    c = c_ref[...]
    zh = jnp.dot(h, whh_ref[...], preferred_element_type=jnp.float32) \
        + bhh_ref[...]
    zi = z_ref[:, 0, 0, :] + zh[:, 0:_D]
    zf = z_ref[:, 0, 1, :] + zh[:, _D:2 * _D]
    zg = z_ref[:, 0, 2, :] + zh[:, 2 * _D:3 * _D]
    zo = z_ref[:, 0, 3, :] + zh[:, 3 * _D:]
    sig = lambda v: 1.0 / (1.0 + jnp.exp(-v))
    c_new = sig(zf) * c + sig(zi) * jnp.tanh(zg)
    h_new = sig(zo) * jnp.tanh(c_new)
    h_ref[...] = h_new
    c_ref[...] = c_new
    hp = jnp.dot(h_new, wh_ref[...], preferred_element_type=jnp.float32)
    s_t = scal_ref[0]
    out_ref[...] = sig(hp + s_t).reshape(1, _B, 1)


_LSTM_KW = dict(
    grid=(_S,),
    in_specs=[
        pl.BlockSpec((_B, 1, 4, _D), lambda t: (0, t, 0, 0)),
        pl.BlockSpec((1, _B, 1), lambda t: (t, 0, 0)),
        _FULL((_D, 4 * _D)),
        _FULL((1, 4 * _D)),
        _FULL((_D, 1)),
    ],
    out_specs=pl.BlockSpec((1, _B, 1), lambda t: (t, 0, 0)),
    out_shape=jax.ShapeDtypeStruct((_S, _B, 1), jnp.float32),
    scratch_shapes=[
        pltpu.VMEM((_B, _D), jnp.float32),
        pltpu.VMEM((_B, _D), jnp.float32),
    ],
    compiler_params=pltpu.CompilerParams(dimension_semantics=("arbitrary",)),
)


def kernel(true_length, p, q, r, aff, q_next, p_next, emb_p, emb_q, emb_r,
           emb_aff, W_aff, b_aff, W1, att_src1, att_dst1, b1, W2, att_src2,
           att_dst2, b2, W_ih, W_hh, b_ih, b_hh, W_out, b_out):
    f32 = jnp.float32
    i32 = jnp.int32
    pad = lambda a: jnp.pad(a.astype(i32), ((0, 0), (0, _SP - _S)))

    idx4 = jnp.stack([pad(p), pad(q), pad(q_next), pad(p_next)]) \
        .reshape(4, _NW, _NCH, _CH)
    pe, qe, qn, pn = [g.reshape(_B, _SP, _D)
                      for g in _sc_gather(emb_p.astype(f32), emb_q.astype(f32),
                                          idx4)]

    aff4 = pad(aff)[..., None]
    r4 = pad(r)[..., None]

    Wa = W_aff.T
    As1 = (att_src1[:, :, None] * jnp.eye(8, dtype=f32)[:, None, :]) \
        .reshape(8 * 128, 8)
    Ad1 = (att_dst1[:, :, None] * jnp.eye(8, dtype=f32)[:, None, :]) \
        .reshape(8 * 128, 8)
    E8 = jnp.kron(jnp.eye(8, dtype=f32), jnp.ones((1, 128), f32))
    embaff16 = jnp.pad(emb_aff, ((0, 5), (0, 0)))
    wqn = W_out[0, _D:2 * _D][:, None]
    wpn = W_out[0, 2 * _D:][:, None]
    wh = W_out[0, :_D][:, None]

    zx, scal3 = pl.pallas_call(_prelude_body, **_PRELUDE_KW)(
        true_length.astype(i32), b_out.astype(f32),
        pe, qe, qn, pn, aff4, r4,
        Wa, b_aff[None], W1.T, As1, Ad1, E8, b1[None],
        W2.T, att_src2.T, att_dst2.T, b2[None],
        W_ih.T, b_ih[None], emb_r, embaff16, wqn, wpn)

    out_tb = pl.pallas_call(_lstm_body, **_LSTM_KW)(
        zx, jnp.transpose(scal3, (1, 0, 2)), W_hh.T, b_hh[None], wh)

    return jnp.transpose(out_tb, (1, 0, 2))


# speedup vs baseline: 1.4235x; 1.1563x over previous
import functools

import jax
import jax.numpy as jnp
from jax import lax
from jax.experimental import pallas as pl
from jax.experimental.pallas import tpu as pltpu
from jax.experimental.pallas import tpu_sc as plsc

_B, _S, _D = 64, 499, 256
_SP = 512
_NW = 32
_RPW = (_B * _SP) // _NW
_CH = 128
_NCH = _RPW // _CH


def _sc_gather_body(emb_p_h, emb_q_h, idx_h, o0, o1, o2, o3,
                    idx_v, rows0, rows1, gs0, gs1, ws0, ws1):
    wid = lax.axis_index("s") * 2 + lax.axis_index("c")
    base = wid * _RPW
    tabs = (emb_p_h, emb_q_h, emb_q_h, emb_p_h)
    outs = (o0, o1, o2, o3)
    for t in range(4):
        pltpu.sync_copy(idx_h.at[t, wid], idx_v.at[t])
    bufs = (rows0, rows1)
    gsem = (gs0, gs1)
    wsem = (ws0, ws1)
    seq = [(t, c) for t in range(4) for c in range(_NCH)]
    n = len(seq)
    gcp = [None, None]
    wcp = [None, None]
    gcp[0] = pltpu.async_copy(tabs[0].at[idx_v.at[0, 0]], bufs[0], gs0)
    for k in range(n):
        s = k % 2
        t, c = seq[k]
        gcp[s].wait()
        if k + 1 < n:
            s2 = (k + 1) % 2
            t2, c2 = seq[k + 1]
            if wcp[s2] is not None:
                wcp[s2].wait()
            gcp[s2] = pltpu.async_copy(tabs[t2].at[idx_v.at[t2, c2]],
                                       bufs[s2], gsem[s2])
        wcp[s] = pltpu.async_copy(
            bufs[s], outs[t].at[pl.ds(base + c * _CH, _CH)], wsem[s])
    wcp[0].wait()
    wcp[1].wait()


def _sc_gather(emb_p, emb_q, idx4):
    row = jax.ShapeDtypeStruct((_B * _SP, _D), jnp.float32)
    mesh = plsc.VectorSubcoreMesh(core_axis_name="c", subcore_axis_name="s")
    f = functools.partial(
        pl.kernel,
        out_type=[row, row, row, row],
        mesh=mesh,
        scratch_types=[
            pltpu.VMEM((4, _NCH, _CH), jnp.int32),
            pltpu.VMEM((_CH, _D), jnp.float32),
            pltpu.VMEM((_CH, _D), jnp.float32),
            pltpu.SemaphoreType.DMA,
            pltpu.SemaphoreType.DMA,
            pltpu.SemaphoreType.DMA,
            pltpu.SemaphoreType.DMA,
        ],
    )(_sc_gather_body)
    return f(emb_p, emb_q, idx4)


def _prelude_body(tl_ref, bout_ref,
                  pe_ref, qe_ref, qn_ref, pn_ref, aff_ref, r_ref,
                  Wa_ref, baff_ref, W1T_ref, As1_ref, Ad1_ref, E8_ref, b1_ref,
                  W2T_ref, as2_ref, ad2_ref, b2_ref,
                  WihT_ref, bih_ref, embr_ref, embaff_ref, wqn_ref, wpn_ref,
                  zx_ref, scal_ref):
    b = pl.program_id(0)
    f32 = jnp.float32
    dot = lambda a, w: jnp.dot(a, w, preferred_element_type=f32)

    n = tl_ref[b] - 1
    jcol = lax.broadcasted_iota(jnp.int32, (_SP, 1), 0)
    mp = (jcol >= 1) & (jcol < n)
    mn = (jcol + 1) < n

    pe = pe_ref[0]
    qe = qe_ref[0]

    a_col = aff_ref[0]
    oh = (a_col == lax.broadcasted_iota(jnp.int32, (_SP, 16), 1)).astype(f32)
    affproj = dot(embaff_ref[...], Wa_ref[_D:])
    x = dot(pe, Wa_ref[:_D]) + dot(oh, affproj) + baff_ref[...]

    neg = f32(-1e30)
    lrelu = lambda v: jnp.where(v >= 0, v, 0.2 * v)

    def stencil(hmat, a_s, a_d, nlanes):
        zrow = jnp.zeros((1, nlanes), f32)
        a_sp = jnp.concatenate([jnp.zeros((1, a_s.shape[1]), f32), a_s[:-1]], 0)
        a_sn = jnp.concatenate([a_s[1:], jnp.zeros((1, a_s.shape[1]), f32)], 0)
        al_s = lrelu(a_s + a_d)
        al_p = lrelu(a_sp + a_d)
        al_n = lrelu(a_sn + a_d)
        amax = jnp.maximum(al_s, jnp.maximum(jnp.where(mp, al_p, neg),
                                             jnp.where(mn, al_n, neg)))
        e_s = jnp.exp(al_s - amax)
        e_p = jnp.where(mp, jnp.exp(al_p - amax), 0.0)
        e_n = jnp.where(mn, jnp.exp(al_n - amax), 0.0)
        den = e_s + e_p + e_n + 1e-16
        h_p = jnp.concatenate([zrow, hmat[:-1]], 0)
        h_n = jnp.concatenate([hmat[1:], zrow], 0)
        return (e_s / den), (e_p / den), (e_n / den), h_p, h_n

    h1 = dot(x, W1T_ref[...])
    a_s1 = dot(h1, As1_ref[...])
    a_d1 = dot(h1, Ad1_ref[...])
    cs, cp, cn, h1p, h1n = stencil(h1, a_s1, a_d1, 8 * 128)
    g1 = (dot(cs, E8_ref[...]) * h1 + dot(cp, E8_ref[...]) * h1p
          + dot(cn, E8_ref[...]) * h1n + b1_ref[...])
    x1 = jnp.where(g1 > 0, g1, jnp.exp(jnp.minimum(g1, 0.0)) - 1.0)

    h2 = dot(x1, W2T_ref[...])
    a_s2 = dot(h2, as2_ref[...])
    a_d2 = dot(h2, ad2_ref[...])
    cs2, cp2, cn2, h2p, h2n = stencil(h2, a_s2, a_d2, _D)
    aff_out = cs2 * h2 + cp2 * h2p + cn2 * h2n + b2_ref[...]

    z = (dot(pe, WihT_ref[:_D]) + dot(qe, WihT_ref[_D:2 * _D])
         + dot(aff_out, WihT_ref[3 * _D:]) + bih_ref[...])
    zr = dot(embr_ref[...], WihT_ref[2 * _D:3 * _D])
    rf = r_ref[0].astype(f32)
    z = z + zr[0:1] + rf * (zr[1:2] - zr[0:1])
    for k in range(4):
        zx_ref[0, :, k, :] = z[:, k * _D:(k + 1) * _D]

    scal_ref[0] = dot(qn_ref[0], wqn_ref[...]) + dot(pn_ref[0], wpn_ref[...]) \
        + bout_ref[0]


_FULL = lambda shp: pl.BlockSpec(shp, lambda b: (0,) * len(shp))
_ROW = lambda shp: pl.BlockSpec(shp, lambda b: (b,) + (0,) * (len(shp) - 1))

_PRELUDE_KW = dict(
    grid=(_B,),
    in_specs=[
        pl.BlockSpec(memory_space=pltpu.SMEM),
        pl.BlockSpec(memory_space=pltpu.SMEM),
        _ROW((1, _SP, _D)),
        _ROW((1, _SP, _D)),
        _ROW((1, _SP, _D)),
        _ROW((1, _SP, _D)),
        _ROW((1, _SP, 1)),
        _ROW((1, _SP, 1)),
        _FULL((2 * _D, _D)),
        _FULL((1, _D)),
        _FULL((_D, 8 * 128)),
        _FULL((8 * 128, 8)),
        _FULL((8 * 128, 8)),
        _FULL((8, 8 * 128)),
        _FULL((1, 8 * 128)),
        _FULL((8 * 128, _D)),
        _FULL((_D, 1)),
        _FULL((_D, 1)),
        _FULL((1, _D)),
        _FULL((4 * _D, 4 * _D)),
        _FULL((1, 4 * _D)),
        _FULL((2, _D)),
        _FULL((16, _D)),
        _FULL((_D, 1)),
        _FULL((_D, 1)),
    ],
    out_specs=[
        pl.BlockSpec((1, _SP, 4, _D), lambda b: (b, 0, 0, 0)),
        pl.BlockSpec((1, _SP, 1), lambda b: (b, 0, 0)),
    ],
    out_shape=[
        jax.ShapeDtypeStruct((_B, _SP, 4, _D), jnp.float32),
        jax.ShapeDtypeStruct((_B, _SP, 1), jnp.float32),
    ],
    compiler_params=pltpu.CompilerParams(dimension_semantics=("arbitrary",)),
)


_UT = 4


def _lstm_body(z_ref, scal_ref, whh_ref, bhh_ref, wh_ref, out_ref, h_ref, c_ref):
    t = pl.program_id(0)

    @pl.when(t == 0)
    def _init():
        h_ref[...] = jnp.zeros((_B, _D), jnp.float32)
        c_ref[...] = jnp.zeros((_B, _D), jnp.float32)

    h = h_ref[...]
    c = c_ref[...]
    sig = lambda v: 1.0 / (1.0 + jnp.exp(-v))
    for u in range(_UT):
        zh = jnp.dot(h, whh_ref[...], preferred_element_type=jnp.float32) \
            + bhh_ref[...]
        zi = z_ref[:, u, 0, :] + zh[:, 0:_D]
        zf = z_ref[:, u, 1, :] + zh[:, _D:2 * _D]
        zg = z_ref[:, u, 2, :] + zh[:, 2 * _D:3 * _D]
        zo = z_ref[:, u, 3, :] + zh[:, 3 * _D:]
        c = sig(zf) * c + sig(zi) * jnp.tanh(zg)
        h = sig(zo) * jnp.tanh(c)
        hp = jnp.dot(h, wh_ref[...], preferred_element_type=jnp.float32)
        out_ref[u] = hp + scal_ref[u]
    h_ref[...] = h
    c_ref[...] = c
    out_ref[...] = sig(out_ref[...])


_LSTM_KW = dict(
    grid=(_SP // _UT,),
    in_specs=[
        pl.BlockSpec((_B, _UT, 4, _D), lambda t: (0, t, 0, 0)),
        pl.BlockSpec((_UT, _B, 1), lambda t: (t, 0, 0)),
        _FULL((_D, 4 * _D)),
        _FULL((1, 4 * _D)),
        _FULL((_D, 1)),
    ],
    out_specs=pl.BlockSpec((_UT, _B, 1), lambda t: (t, 0, 0)),
    out_shape=jax.ShapeDtypeStruct((_SP, _B, 1), jnp.float32),
    scratch_shapes=[
        pltpu.VMEM((_B, _D), jnp.float32),
        pltpu.VMEM((_B, _D), jnp.float32),
    ],
    compiler_params=pltpu.CompilerParams(dimension_semantics=("arbitrary",)),
)


def kernel(true_length, p, q, r, aff, q_next, p_next, emb_p, emb_q, emb_r,
           emb_aff, W_aff, b_aff, W1, att_src1, att_dst1, b1, W2, att_src2,
           att_dst2, b2, W_ih, W_hh, b_ih, b_hh, W_out, b_out):
    f32 = jnp.float32
    i32 = jnp.int32
    pad = lambda a: jnp.pad(a.astype(i32), ((0, 0), (0, _SP - _S)))

    idx4 = jnp.stack([pad(p), pad(q), pad(q_next), pad(p_next)]) \
        .reshape(4, _NW, _NCH, _CH)
    pe, qe, qn, pn = [g.reshape(_B, _SP, _D)
                      for g in _sc_gather(emb_p.astype(f32), emb_q.astype(f32),
                                          idx4)]

    aff4 = pad(aff)[..., None]
    r4 = pad(r)[..., None]

    Wa = W_aff.T
    As1 = (att_src1[:, :, None] * jnp.eye(8, dtype=f32)[:, None, :]) \
        .reshape(8 * 128, 8)
    Ad1 = (att_dst1[:, :, None] * jnp.eye(8, dtype=f32)[:, None, :]) \
        .reshape(8 * 128, 8)
    E8 = jnp.kron(jnp.eye(8, dtype=f32), jnp.ones((1, 128), f32))
    embaff16 = jnp.pad(emb_aff, ((0, 5), (0, 0)))
    wqn = W_out[0, _D:2 * _D][:, None]
    wpn = W_out[0, 2 * _D:][:, None]
    wh = W_out[0, :_D][:, None]

    zx, scal3 = pl.pallas_call(_prelude_body, **_PRELUDE_KW)(
        true_length.astype(i32), b_out.astype(f32),
        pe, qe, qn, pn, aff4, r4,
        Wa, b_aff[None], W1.T, As1, Ad1, E8, b1[None],
        W2.T, att_src2.T, att_dst2.T, b2[None],
        W_ih.T, b_ih[None], emb_r, embaff16, wqn, wpn)

    out_tb = pl.pallas_call(_lstm_body, **_LSTM_KW)(
        zx, jnp.transpose(scal3, (1, 0, 2)), W_hh.T, b_hh[None], wh)

    return jnp.transpose(out_tb[:_S], (1, 0, 2))


# speedup vs baseline: 1.4560x; 1.0228x over previous
import functools

import jax
import jax.numpy as jnp
from jax import lax
from jax.experimental import pallas as pl
from jax.experimental.pallas import tpu as pltpu
from jax.experimental.pallas import tpu_sc as plsc

_B, _S, _D = 64, 499, 256
_SP = 512
_NW = 32
_RPW = (_B * _SP) // _NW
_CH = 128
_NCH = _RPW // _CH


def _sc_gather_body(emb_p_h, emb_q_h, idx_h, o0, o1, o2, o3,
                    idx_v, rows0, rows1, gs0, gs1, ws0, ws1):
    wid = lax.axis_index("s") * 2 + lax.axis_index("c")
    base = wid * _RPW
    tabs = (emb_p_h, emb_q_h, emb_q_h, emb_p_h)
    outs = (o0, o1, o2, o3)
    for t in range(4):
        pltpu.sync_copy(idx_h.at[t, wid], idx_v.at[t])
    bufs = (rows0, rows1)
    gsem = (gs0, gs1)
    wsem = (ws0, ws1)
    seq = [(t, c) for t in range(4) for c in range(_NCH)]
    n = len(seq)
    gcp = [None, None]
    wcp = [None, None]
    gcp[0] = pltpu.async_copy(tabs[0].at[idx_v.at[0, 0]], bufs[0], gs0)
    for k in range(n):
        s = k % 2
        t, c = seq[k]
        gcp[s].wait()
        if k + 1 < n:
            s2 = (k + 1) % 2
            t2, c2 = seq[k + 1]
            if wcp[s2] is not None:
                wcp[s2].wait()
            gcp[s2] = pltpu.async_copy(tabs[t2].at[idx_v.at[t2, c2]],
                                       bufs[s2], gsem[s2])
        wcp[s] = pltpu.async_copy(
            bufs[s], outs[t].at[pl.ds(base + c * _CH, _CH)], wsem[s])
    wcp[0].wait()
    wcp[1].wait()


def _sc_gather(emb_p, emb_q, idx4):
    row = jax.ShapeDtypeStruct((_B * _SP, _D), jnp.float32)
    mesh = plsc.VectorSubcoreMesh(core_axis_name="c", subcore_axis_name="s")
    f = functools.partial(
        pl.kernel,
        out_type=[row, row, row, row],
        mesh=mesh,
        scratch_types=[
            pltpu.VMEM((4, _NCH, _CH), jnp.int32),
            pltpu.VMEM((_CH, _D), jnp.float32),
            pltpu.VMEM((_CH, _D), jnp.float32),
            pltpu.SemaphoreType.DMA,
            pltpu.SemaphoreType.DMA,
            pltpu.SemaphoreType.DMA,
            pltpu.SemaphoreType.DMA,
        ],
    )(_sc_gather_body)
    return f(emb_p, emb_q, idx4)


def _prelude_body(tl_ref, bout_ref,
                  pe_ref, qe_ref, qn_ref, pn_ref, aff_ref, r_ref,
                  Wa_ref, baff_ref, W1T_ref, As1_ref, Ad1_ref, E8_ref, b1_ref,
                  W2T_ref, as2_ref, ad2_ref, b2_ref,
                  WihT_ref, bih_ref, embr_ref, embaff_ref, wqn_ref, wpn_ref,
                  zx_ref, scal_ref):
    b = pl.program_id(0)
    f32 = jnp.float32
    dot = lambda a, w: jnp.dot(a, w, preferred_element_type=f32)

    n = tl_ref[b] - 1
    jcol = lax.broadcasted_iota(jnp.int32, (_SP, 1), 0)
    mp = (jcol >= 1) & (jcol < n)
    mn = (jcol + 1) < n

    pe = pe_ref[0]
    qe = qe_ref[0]

    a_col = aff_ref[0]
    oh = (a_col == lax.broadcasted_iota(jnp.int32, (_SP, 16), 1)).astype(f32)
    affproj = dot(embaff_ref[...], Wa_ref[_D:])
    x = dot(pe, Wa_ref[:_D]) + dot(oh, affproj) + baff_ref[...]

    neg = f32(-1e30)
    lrelu = lambda v: jnp.where(v >= 0, v, 0.2 * v)

    def stencil(hmat, a_s, a_d, nlanes):
        zrow = jnp.zeros((1, nlanes), f32)
        a_sp = jnp.concatenate([jnp.zeros((1, a_s.shape[1]), f32), a_s[:-1]], 0)
        a_sn = jnp.concatenate([a_s[1:], jnp.zeros((1, a_s.shape[1]), f32)], 0)
        al_s = lrelu(a_s + a_d)
        al_p = lrelu(a_sp + a_d)
        al_n = lrelu(a_sn + a_d)
        amax = jnp.maximum(al_s, jnp.maximum(jnp.where(mp, al_p, neg),
                                             jnp.where(mn, al_n, neg)))
        e_s = jnp.exp(al_s - amax)
        e_p = jnp.where(mp, jnp.exp(al_p - amax), 0.0)
        e_n = jnp.where(mn, jnp.exp(al_n - amax), 0.0)
        den = e_s + e_p + e_n + 1e-16
        h_p = jnp.concatenate([zrow, hmat[:-1]], 0)
        h_n = jnp.concatenate([hmat[1:], zrow], 0)
        return (e_s / den), (e_p / den), (e_n / den), h_p, h_n

    h1 = dot(x, W1T_ref[...])
    a_s1 = dot(h1, As1_ref[...])
    a_d1 = dot(h1, Ad1_ref[...])
    cs, cp, cn, h1p, h1n = stencil(h1, a_s1, a_d1, 8 * 128)
    g1 = (dot(cs, E8_ref[...]) * h1 + dot(cp, E8_ref[...]) * h1p
          + dot(cn, E8_ref[...]) * h1n + b1_ref[...])
    x1 = jnp.where(g1 > 0, g1, jnp.exp(jnp.minimum(g1, 0.0)) - 1.0)

    h2 = dot(x1, W2T_ref[...])
    a_s2 = dot(h2, as2_ref[...])
    a_d2 = dot(h2, ad2_ref[...])
    cs2, cp2, cn2, h2p, h2n = stencil(h2, a_s2, a_d2, _D)
    aff_out = cs2 * h2 + cp2 * h2p + cn2 * h2n + b2_ref[...]

    z = (dot(pe, WihT_ref[:_D]) + dot(qe, WihT_ref[_D:2 * _D])
         + dot(aff_out, WihT_ref[3 * _D:]) + bih_ref[...])
    zr = dot(embr_ref[...], WihT_ref[2 * _D:3 * _D])
    rf = r_ref[0].astype(f32)
    z = z + zr[0:1] + rf * (zr[1:2] - zr[0:1])
    for k in range(4):
        zx_ref[0, :, k, :] = z[:, k * _D:(k + 1) * _D]

    scal_ref[0] = dot(qn_ref[0], wqn_ref[...]) + dot(pn_ref[0], wpn_ref[...]) \
        + bout_ref[0]


_FULL = lambda shp: pl.BlockSpec(shp, lambda b: (0,) * len(shp))
_ROW = lambda shp: pl.BlockSpec(shp, lambda b: (b,) + (0,) * (len(shp) - 1))

_PRELUDE_KW = dict(
    grid=(_B,),
    in_specs=[
        pl.BlockSpec(memory_space=pltpu.SMEM),
        pl.BlockSpec(memory_space=pltpu.SMEM),
        _ROW((1, _SP, _D)),
        _ROW((1, _SP, _D)),
        _ROW((1, _SP, _D)),
        _ROW((1, _SP, _D)),
        _ROW((1, _SP, 1)),
        _ROW((1, _SP, 1)),
        _FULL((2 * _D, _D)),
        _FULL((1, _D)),
        _FULL((_D, 8 * 128)),
        _FULL((8 * 128, 8)),
        _FULL((8 * 128, 8)),
        _FULL((8, 8 * 128)),
        _FULL((1, 8 * 128)),
        _FULL((8 * 128, _D)),
        _FULL((_D, 1)),
        _FULL((_D, 1)),
        _FULL((1, _D)),
        _FULL((4 * _D, 4 * _D)),
        _FULL((1, 4 * _D)),
        _FULL((2, _D)),
        _FULL((16, _D)),
        _FULL((_D, 1)),
        _FULL((_D, 1)),
    ],
    out_specs=[
        pl.BlockSpec((1, _SP, 4, _D), lambda b: (b, 0, 0, 0)),
        pl.BlockSpec((1, _SP, 1), lambda b: (b, 0, 0)),
    ],
    out_shape=[
        jax.ShapeDtypeStruct((_B, _SP, 4, _D), jnp.float32),
        jax.ShapeDtypeStruct((_B, _SP, 1), jnp.float32),
    ],
    compiler_params=pltpu.CompilerParams(dimension_semantics=("arbitrary",)),
)


_UT = 8


def _lstm_body(z_ref, scal_ref, whh_ref, bhh_ref, wh_ref, out_ref, h_ref, c_ref):
    t = pl.program_id(0)

    @pl.when(t == 0)
    def _init():
        h_ref[...] = jnp.zeros((_B, _D), jnp.float32)
        c_ref[...] = jnp.zeros((_B, _D), jnp.float32)

    h = h_ref[...]
    c = c_ref[...]
    sig = lambda v: 1.0 / (1.0 + jnp.exp(-v))
    for u in range(_UT):
        zh = jnp.dot(h, whh_ref[...], preferred_element_type=jnp.float32) \
            + bhh_ref[...]
        zi = z_ref[:, u, 0, :] + zh[:, 0:_D]
        zf = z_ref[:, u, 1, :] + zh[:, _D:2 * _D]
        zg = z_ref[:, u, 2, :] + zh[:, 2 * _D:3 * _D]
        zo = z_ref[:, u, 3, :] + zh[:, 3 * _D:]
        c = sig(zf) * c + sig(zi) * jnp.tanh(zg)
        h = sig(zo) * jnp.tanh(c)
        hp = jnp.dot(h, wh_ref[...], preferred_element_type=jnp.float32)
        out_ref[u] = hp + scal_ref[u]
    h_ref[...] = h
    c_ref[...] = c
    out_ref[...] = sig(out_ref[...])


_LSTM_KW = dict(
    grid=(_SP // _UT,),
    in_specs=[
        pl.BlockSpec((_B, _UT, 4, _D), lambda t: (0, t, 0, 0)),
        pl.BlockSpec((_UT, _B, 1), lambda t: (t, 0, 0)),
        _FULL((_D, 4 * _D)),
        _FULL((1, 4 * _D)),
        _FULL((_D, 1)),
    ],
    out_specs=pl.BlockSpec((_UT, _B, 1), lambda t: (t, 0, 0)),
    out_shape=jax.ShapeDtypeStruct((_SP, _B, 1), jnp.float32),
    scratch_shapes=[
        pltpu.VMEM((_B, _D), jnp.float32),
        pltpu.VMEM((_B, _D), jnp.float32),
    ],
    compiler_params=pltpu.CompilerParams(dimension_semantics=("arbitrary",)),
)


def kernel(true_length, p, q, r, aff, q_next, p_next, emb_p, emb_q, emb_r,
           emb_aff, W_aff, b_aff, W1, att_src1, att_dst1, b1, W2, att_src2,
           att_dst2, b2, W_ih, W_hh, b_ih, b_hh, W_out, b_out):
    f32 = jnp.float32
    i32 = jnp.int32
    pad = lambda a: jnp.pad(a.astype(i32), ((0, 0), (0, _SP - _S)))

    idx4 = jnp.stack([pad(p), pad(q), pad(q_next), pad(p_next)]) \
        .reshape(4, _NW, _NCH, _CH)
    pe, qe, qn, pn = [g.reshape(_B, _SP, _D)
                      for g in _sc_gather(emb_p.astype(f32), emb_q.astype(f32),
                                          idx4)]

    aff4 = pad(aff)[..., None]
    r4 = pad(r)[..., None]

    Wa = W_aff.T
    As1 = (att_src1[:, :, None] * jnp.eye(8, dtype=f32)[:, None, :]) \
        .reshape(8 * 128, 8)
    Ad1 = (att_dst1[:, :, None] * jnp.eye(8, dtype=f32)[:, None, :]) \
        .reshape(8 * 128, 8)
    E8 = jnp.kron(jnp.eye(8, dtype=f32), jnp.ones((1, 128), f32))
    embaff16 = jnp.pad(emb_aff, ((0, 5), (0, 0)))
    wqn = W_out[0, _D:2 * _D][:, None]
    wpn = W_out[0, 2 * _D:][:, None]
    wh = W_out[0, :_D][:, None]

    zx, scal3 = pl.pallas_call(_prelude_body, **_PRELUDE_KW)(
        true_length.astype(i32), b_out.astype(f32),
        pe, qe, qn, pn, aff4, r4,
        Wa, b_aff[None], W1.T, As1, Ad1, E8, b1[None],
        W2.T, att_src2.T, att_dst2.T, b2[None],
        W_ih.T, b_ih[None], emb_r, embaff16, wqn, wpn)

    out_tb = pl.pallas_call(_lstm_body, **_LSTM_KW)(
        zx, jnp.transpose(scal3, (1, 0, 2)), W_hh.T, b_hh[None], wh)

    return jnp.transpose(out_tb[:_S], (1, 0, 2))


# speedup vs baseline: 1.4656x; 1.0066x over previous
import functools

import jax
import jax.numpy as jnp
from jax import lax
from jax.experimental import pallas as pl
from jax.experimental.pallas import tpu as pltpu
from jax.experimental.pallas import tpu_sc as plsc

_B, _S, _D = 64, 499, 256
_SP = 512
_NW = 32
_RPW = (_B * _SP) // _NW
_CH = 128
_NCH = _RPW // _CH


def _sc_gather_body(emb_p_h, emb_q_h, idx_h, o0, o1, o2, o3,
                    idx_v, rows0, rows1, gs0, gs1, ws0, ws1):
    wid = lax.axis_index("s") * 2 + lax.axis_index("c")
    base = wid * _RPW
    tabs = (emb_p_h, emb_q_h, emb_q_h, emb_p_h)
    outs = (o0, o1, o2, o3)
    for t in range(4):
        pltpu.sync_copy(idx_h.at[t, wid], idx_v.at[t])
    bufs = (rows0, rows1)
    gsem = (gs0, gs1)
    wsem = (ws0, ws1)
    seq = [(t, c) for t in range(4) for c in range(_NCH)]
    n = len(seq)
    gcp = [None, None]
    wcp = [None, None]
    gcp[0] = pltpu.async_copy(tabs[0].at[idx_v.at[0, 0]], bufs[0], gs0)
    for k in range(n):
        s = k % 2
        t, c = seq[k]
        gcp[s].wait()
        if k + 1 < n:
            s2 = (k + 1) % 2
            t2, c2 = seq[k + 1]
            if wcp[s2] is not None:
                wcp[s2].wait()
            gcp[s2] = pltpu.async_copy(tabs[t2].at[idx_v.at[t2, c2]],
                                       bufs[s2], gsem[s2])
        wcp[s] = pltpu.async_copy(
            bufs[s], outs[t].at[pl.ds(base + c * _CH, _CH)], wsem[s])
    wcp[0].wait()
    wcp[1].wait()


def _sc_gather(emb_p, emb_q, idx4):
    row = jax.ShapeDtypeStruct((_B * _SP, _D), jnp.float32)
    mesh = plsc.VectorSubcoreMesh(core_axis_name="c", subcore_axis_name="s")
    f = functools.partial(
        pl.kernel,
        out_type=[row, row, row, row],
        mesh=mesh,
        scratch_types=[
            pltpu.VMEM((4, _NCH, _CH), jnp.int32),
            pltpu.VMEM((_CH, _D), jnp.float32),
            pltpu.VMEM((_CH, _D), jnp.float32),
            pltpu.SemaphoreType.DMA,
            pltpu.SemaphoreType.DMA,
            pltpu.SemaphoreType.DMA,
            pltpu.SemaphoreType.DMA,
        ],
    )(_sc_gather_body)
    return f(emb_p, emb_q, idx4)


def _prelude_body(tl_ref, bout_ref,
                  pe_ref, qe_ref, qn_ref, pn_ref, aff_ref, r_ref,
                  Wa_ref, baff_ref, W1T_ref, As1_ref, Ad1_ref, E8_ref, b1_ref,
                  W2T_ref, as2_ref, ad2_ref, b2_ref,
                  WihT_ref, bih_ref, embr_ref, embaff_ref, wqn_ref, wpn_ref,
                  zx_ref, scal_ref):
    b = pl.program_id(0)
    f32 = jnp.float32
    dot = lambda a, w: jnp.dot(a, w, preferred_element_type=f32)

    n = tl_ref[b] - 1
    jcol = lax.broadcasted_iota(jnp.int32, (_SP, 1), 0)
    mp = (jcol >= 1) & (jcol < n)
    mn = (jcol + 1) < n

    pe = pe_ref[0]
    qe = qe_ref[0]

    a_col = aff_ref[0]
    oh = (a_col == lax.broadcasted_iota(jnp.int32, (_SP, 16), 1)).astype(f32)
    affproj = dot(embaff_ref[...], Wa_ref[_D:])
    x = dot(pe, Wa_ref[:_D]) + dot(oh, affproj) + baff_ref[...]

    neg = f32(-1e30)
    lrelu = lambda v: jnp.where(v >= 0, v, 0.2 * v)

    def stencil(hmat, a_s, a_d, nlanes):
        zrow = jnp.zeros((1, nlanes), f32)
        a_sp = jnp.concatenate([jnp.zeros((1, a_s.shape[1]), f32), a_s[:-1]], 0)
        a_sn = jnp.concatenate([a_s[1:], jnp.zeros((1, a_s.shape[1]), f32)], 0)
        al_s = lrelu(a_s + a_d)
        al_p = lrelu(a_sp + a_d)
        al_n = lrelu(a_sn + a_d)
        amax = jnp.maximum(al_s, jnp.maximum(jnp.where(mp, al_p, neg),
                                             jnp.where(mn, al_n, neg)))
        e_s = jnp.exp(al_s - amax)
        e_p = jnp.where(mp, jnp.exp(al_p - amax), 0.0)
        e_n = jnp.where(mn, jnp.exp(al_n - amax), 0.0)
        den = e_s + e_p + e_n + 1e-16
        h_p = jnp.concatenate([zrow, hmat[:-1]], 0)
        h_n = jnp.concatenate([hmat[1:], zrow], 0)
        return (e_s / den), (e_p / den), (e_n / den), h_p, h_n

    h1 = dot(x, W1T_ref[...])
    a_s1 = dot(h1, As1_ref[...])
    a_d1 = dot(h1, Ad1_ref[...])
    cs, cp, cn, h1p, h1n = stencil(h1, a_s1, a_d1, 8 * 128)
    g1 = (dot(cs, E8_ref[...]) * h1 + dot(cp, E8_ref[...]) * h1p
          + dot(cn, E8_ref[...]) * h1n + b1_ref[...])
    x1 = jnp.where(g1 > 0, g1, jnp.exp(jnp.minimum(g1, 0.0)) - 1.0)

    h2 = dot(x1, W2T_ref[...])
    a_s2 = dot(h2, as2_ref[...])
    a_d2 = dot(h2, ad2_ref[...])
    cs2, cp2, cn2, h2p, h2n = stencil(h2, a_s2, a_d2, _D)
    aff_out = cs2 * h2 + cp2 * h2p + cn2 * h2n + b2_ref[...]

    z = (dot(pe, WihT_ref[:_D]) + dot(qe, WihT_ref[_D:2 * _D])
         + dot(aff_out, WihT_ref[3 * _D:]) + bih_ref[...])
    zr = dot(embr_ref[...], WihT_ref[2 * _D:3 * _D])
    rf = r_ref[0].astype(f32)
    z = z + zr[0:1] + rf * (zr[1:2] - zr[0:1])
    for k in range(4):
        zx_ref[0, :, k, :] = z[:, k * _D:(k + 1) * _D]

    scal_ref[0] = dot(qn_ref[0], wqn_ref[...]) + dot(pn_ref[0], wpn_ref[...]) \
        + bout_ref[0]


_FULL = lambda shp: pl.BlockSpec(shp, lambda b: (0,) * len(shp))
_ROW = lambda shp: pl.BlockSpec(shp, lambda b: (b,) + (0,) * (len(shp) - 1))

_PRELUDE_KW = dict(
    grid=(_B,),
    in_specs=[
        pl.BlockSpec(memory_space=pltpu.SMEM),
        pl.BlockSpec(memory_space=pltpu.SMEM),
        _ROW((1, _SP, _D)),
        _ROW((1, _SP, _D)),
        _ROW((1, _SP, _D)),
        _ROW((1, _SP, _D)),
        _ROW((1, _SP, 1)),
        _ROW((1, _SP, 1)),
        _FULL((2 * _D, _D)),
        _FULL((1, _D)),
        _FULL((_D, 8 * 128)),
        _FULL((8 * 128, 8)),
        _FULL((8 * 128, 8)),
        _FULL((8, 8 * 128)),
        _FULL((1, 8 * 128)),
        _FULL((8 * 128, _D)),
        _FULL((_D, 1)),
        _FULL((_D, 1)),
        _FULL((1, _D)),
        _FULL((4 * _D, 4 * _D)),
        _FULL((1, 4 * _D)),
        _FULL((2, _D)),
        _FULL((16, _D)),
        _FULL((_D, 1)),
        _FULL((_D, 1)),
    ],
    out_specs=[
        pl.BlockSpec((1, _SP, 4, _D), lambda b: (b, 0, 0, 0)),
        pl.BlockSpec((1, _SP, 1), lambda b: (b, 0, 0)),
    ],
    out_shape=[
        jax.ShapeDtypeStruct((_B, _SP, 4, _D), jnp.float32),
        jax.ShapeDtypeStruct((_B, _SP, 1), jnp.float32),
    ],
    compiler_params=pltpu.CompilerParams(dimension_semantics=("arbitrary",)),
)


_UT = 16


def _lstm_body(z_ref, scal_ref, whh_ref, bhh_ref, wh_ref, out_ref, h_ref, c_ref):
    t = pl.program_id(0)

    @pl.when(t == 0)
    def _init():
        h_ref[...] = jnp.zeros((_B, _D), jnp.float32)
        c_ref[...] = jnp.zeros((_B, _D), jnp.float32)

    h = h_ref[...]
    c = c_ref[...]
    sig = lambda v: 1.0 / (1.0 + jnp.exp(-v))
    for u in range(_UT):
        zh = jnp.dot(h, whh_ref[...], preferred_element_type=jnp.float32) \
            + bhh_ref[...]
        zi = z_ref[:, u, 0, :] + zh[:, 0:_D]
        zf = z_ref[:, u, 1, :] + zh[:, _D:2 * _D]
        zg = z_ref[:, u, 2, :] + zh[:, 2 * _D:3 * _D]
        zo = z_ref[:, u, 3, :] + zh[:, 3 * _D:]
        c = sig(zf) * c + sig(zi) * jnp.tanh(zg)
        h = sig(zo) * jnp.tanh(c)
        hp = jnp.dot(h, wh_ref[...], preferred_element_type=jnp.float32)
        out_ref[u] = hp + scal_ref[u]
    h_ref[...] = h
    c_ref[...] = c
    out_ref[...] = sig(out_ref[...])


_LSTM_KW = dict(
    grid=(_SP // _UT,),
    in_specs=[
        pl.BlockSpec((_B, _UT, 4, _D), lambda t: (0, t, 0, 0)),
        pl.BlockSpec((_UT, _B, 1), lambda t: (t, 0, 0)),
        _FULL((_D, 4 * _D)),
        _FULL((1, 4 * _D)),
        _FULL((_D, 1)),
    ],
    out_specs=pl.BlockSpec((_UT, _B, 1), lambda t: (t, 0, 0)),
    out_shape=jax.ShapeDtypeStruct((_SP, _B, 1), jnp.float32),
    scratch_shapes=[
        pltpu.VMEM((_B, _D), jnp.float32),
        pltpu.VMEM((_B, _D), jnp.float32),
    ],
    compiler_params=pltpu.CompilerParams(dimension_semantics=("arbitrary",)),
)


def kernel(true_length, p, q, r, aff, q_next, p_next, emb_p, emb_q, emb_r,
           emb_aff, W_aff, b_aff, W1, att_src1, att_dst1, b1, W2, att_src2,
           att_dst2, b2, W_ih, W_hh, b_ih, b_hh, W_out, b_out):
    f32 = jnp.float32
    i32 = jnp.int32
    pad = lambda a: jnp.pad(a.astype(i32), ((0, 0), (0, _SP - _S)))

    idx4 = jnp.stack([pad(p), pad(q), pad(q_next), pad(p_next)]) \
        .reshape(4, _NW, _NCH, _CH)
    pe, qe, qn, pn = [g.reshape(_B, _SP, _D)
                      for g in _sc_gather(emb_p.astype(f32), emb_q.astype(f32),
                                          idx4)]

    aff4 = pad(aff)[..., None]
    r4 = pad(r)[..., None]

    Wa = W_aff.T
    As1 = (att_src1[:, :, None] * jnp.eye(8, dtype=f32)[:, None, :]) \
        .reshape(8 * 128, 8)
    Ad1 = (att_dst1[:, :, None] * jnp.eye(8, dtype=f32)[:, None, :]) \
        .reshape(8 * 128, 8)
    E8 = jnp.kron(jnp.eye(8, dtype=f32), jnp.ones((1, 128), f32))
    embaff16 = jnp.pad(emb_aff, ((0, 5), (0, 0)))
    wqn = W_out[0, _D:2 * _D][:, None]
    wpn = W_out[0, 2 * _D:][:, None]
    wh = W_out[0, :_D][:, None]

    zx, scal3 = pl.pallas_call(_prelude_body, **_PRELUDE_KW)(
        true_length.astype(i32), b_out.astype(f32),
        pe, qe, qn, pn, aff4, r4,
        Wa, b_aff[None], W1.T, As1, Ad1, E8, b1[None],
        W2.T, att_src2.T, att_dst2.T, b2[None],
        W_ih.T, b_ih[None], emb_r, embaff16, wqn, wpn)

    out_tb = pl.pallas_call(_lstm_body, **_LSTM_KW)(
        zx, jnp.transpose(scal3, (1, 0, 2)), W_hh.T, b_hh[None], wh)

    return jnp.transpose(out_tb[:_S], (1, 0, 2))


# speedup vs baseline: 1.5866x; 1.0826x over previous
import functools

import jax
import jax.numpy as jnp
from jax import lax
from jax.experimental import pallas as pl
from jax.experimental.pallas import tpu as pltpu
from jax.experimental.pallas import tpu_sc as plsc

_B, _S, _D = 64, 499, 256
_SP = 512
_NW = 32
_RPW = (_B * _SP) // _NW
_CH = 128
_NCH = _RPW // _CH


def _sc_gather_body(emb_p_h, emb_q_h, idx_h, o0, o1, o2, o3,
                    idx_v, rows0, rows1, gs0, gs1, ws0, ws1):
    wid = lax.axis_index("s") * 2 + lax.axis_index("c")
    base = wid * _RPW
    tabs = (emb_p_h, emb_q_h, emb_q_h, emb_p_h)
    outs = (o0, o1, o2, o3)
    for t in range(4):
        pltpu.sync_copy(idx_h.at[t, wid], idx_v.at[t])
    bufs = (rows0, rows1)
    gsem = (gs0, gs1)
    wsem = (ws0, ws1)
    seq = [(t, c) for t in range(4) for c in range(_NCH)]
    n = len(seq)
    gcp = [None, None]
    wcp = [None, None]
    gcp[0] = pltpu.async_copy(tabs[0].at[idx_v.at[0, 0]], bufs[0], gs0)
    for k in range(n):
        s = k % 2
        t, c = seq[k]
        gcp[s].wait()
        if k + 1 < n:
            s2 = (k + 1) % 2
            t2, c2 = seq[k + 1]
            if wcp[s2] is not None:
                wcp[s2].wait()
            gcp[s2] = pltpu.async_copy(tabs[t2].at[idx_v.at[t2, c2]],
                                       bufs[s2], gsem[s2])
        wcp[s] = pltpu.async_copy(
            bufs[s], outs[t].at[pl.ds(base + c * _CH, _CH)], wsem[s])
    wcp[0].wait()
    wcp[1].wait()


def _sc_gather(emb_p, emb_q, idx4):
    row = jax.ShapeDtypeStruct((_B * _SP, _D), jnp.float32)
    mesh = plsc.VectorSubcoreMesh(core_axis_name="c", subcore_axis_name="s")
    f = functools.partial(
        pl.kernel,
        out_type=[row, row, row, row],
        mesh=mesh,
        scratch_types=[
            pltpu.VMEM((4, _NCH, _CH), jnp.int32),
            pltpu.VMEM((_CH, _D), jnp.float32),
            pltpu.VMEM((_CH, _D), jnp.float32),
            pltpu.SemaphoreType.DMA,
            pltpu.SemaphoreType.DMA,
            pltpu.SemaphoreType.DMA,
            pltpu.SemaphoreType.DMA,
        ],
    )(_sc_gather_body)
    return f(emb_p, emb_q, idx4)


_RB = 2
_M = _RB * _SP


def _prelude_body(tl_ref, bout_ref,
                  pe_ref, qe_ref, qn_ref, pn_ref, aff_ref, r_ref,
                  Wa_ref, baff_ref, W1T_ref, Asd1_ref, E8_ref, b1_ref,
                  W2T_ref, asd2_ref, b2_ref,
                  WihT_ref, bih_ref, embr_ref, embaff_ref, wqn_ref, wpn_ref,
                  zx_ref, scal_ref):
    b = pl.program_id(0)
    f32 = jnp.float32
    dot = lambda a, w: jnp.dot(a, w, preferred_element_type=f32)

    jc = lax.broadcasted_iota(jnp.int32, (_SP, 1), 0)
    mp_l, mn_l = [], []
    for r in range(_RB):
        n = tl_ref[_RB * b + r] - 1
        mp_l.append((jc >= 1) & (jc < n))
        mn_l.append((jc + 1) < n)
    mp = jnp.stack(mp_l)
    mn = jnp.stack(mn_l)

    pe = pe_ref[...].reshape(_M, _D)
    qe = qe_ref[...].reshape(_M, _D)

    a_col = aff_ref[...].reshape(_M, 1)
    oh = (a_col == lax.broadcasted_iota(jnp.int32, (_M, 16), 1)).astype(f32)
    affproj = dot(embaff_ref[...], Wa_ref[_D:])
    x = dot(pe, Wa_ref[:_D]) + dot(oh, affproj) + baff_ref[...]

    neg = f32(-1e30)
    lrelu = lambda v: jnp.where(v >= 0, v, 0.2 * v)

    def stencil(hmat, a_s, a_d, nlanes):
        k = a_s.shape[2]
        zrow = jnp.zeros((_RB, 1, nlanes), f32)
        zk = jnp.zeros((_RB, 1, k), f32)
        a_sp = jnp.concatenate([zk, a_s[:, :-1]], 1)
        a_sn = jnp.concatenate([a_s[:, 1:], zk], 1)
        al_s = lrelu(a_s + a_d)
        al_p = lrelu(a_sp + a_d)
        al_n = lrelu(a_sn + a_d)
        amax = jnp.maximum(al_s, jnp.maximum(jnp.where(mp, al_p, neg),
                                             jnp.where(mn, al_n, neg)))
        e_s = jnp.exp(al_s - amax)
        e_p = jnp.where(mp, jnp.exp(al_p - amax), 0.0)
        e_n = jnp.where(mn, jnp.exp(al_n - amax), 0.0)
        den = e_s + e_p + e_n + 1e-16
        h_p = jnp.concatenate([zrow, hmat[:, :-1]], 1)
        h_n = jnp.concatenate([hmat[:, 1:], zrow], 1)
        return (e_s / den), (e_p / den), (e_n / den), h_p, h_n

    h1 = dot(x, W1T_ref[...])
    asd = dot(h1, Asd1_ref[...])
    h1r = h1.reshape(_RB, _SP, 8 * 128)
    asr = asd.reshape(_RB, _SP, 16)
    cs, cp, cn, h1p, h1n = stencil(h1r, asr[:, :, 0:8], asr[:, :, 8:16],
                                   8 * 128)
    bc = lambda cf: dot(cf.reshape(_M, 8), E8_ref[...]).reshape(
        _RB, _SP, 8 * 128)
    g1 = bc(cs) * h1r + bc(cp) * h1p + bc(cn) * h1n + b1_ref[...]
    x1 = jnp.where(g1 > 0, g1, jnp.exp(jnp.minimum(g1, 0.0)) - 1.0)

    h2 = dot(x1.reshape(_M, 8 * 128), W2T_ref[...])
    a2 = dot(h2, asd2_ref[...])
    h2r = h2.reshape(_RB, _SP, _D)
    a2r = a2.reshape(_RB, _SP, 2)
    cs2, cp2, cn2, h2p, h2n = stencil(h2r, a2r[:, :, 0:1], a2r[:, :, 1:2], _D)
    aff_out = cs2 * h2r + cp2 * h2p + cn2 * h2n + b2_ref[...]

    z = (dot(pe, WihT_ref[:_D]) + dot(qe, WihT_ref[_D:2 * _D])
         + dot(aff_out.reshape(_M, _D), WihT_ref[3 * _D:]) + bih_ref[...])
    zr = dot(embr_ref[...], WihT_ref[2 * _D:3 * _D])
    rf = r_ref[...].reshape(_M, 1).astype(f32)
    z = z + zr[0:1] + rf * (zr[1:2] - zr[0:1])
    zx_ref[...] = z.reshape(_RB, _SP, 4 * _D)

    qn = qn_ref[...].reshape(_M, _D)
    pn = pn_ref[...].reshape(_M, _D)
    scal = dot(qn, wqn_ref[...]) + dot(pn, wpn_ref[...]) + bout_ref[0]
    scal_ref[...] = scal.reshape(_RB, _SP, 1)


_FULL = lambda shp: pl.BlockSpec(shp, lambda b: (0,) * len(shp))
_ROW = lambda shp: pl.BlockSpec(shp, lambda b: (b,) + (0,) * (len(shp) - 1))

_PRELUDE_KW = dict(
    grid=(_B // _RB,),
    in_specs=[
        pl.BlockSpec(memory_space=pltpu.SMEM),
        pl.BlockSpec(memory_space=pltpu.SMEM),
        _ROW((_RB, _SP, _D)),
        _ROW((_RB, _SP, _D)),
        _ROW((_RB, _SP, _D)),
        _ROW((_RB, _SP, _D)),
        _ROW((_RB, _SP, 1)),
        _ROW((_RB, _SP, 1)),
        _FULL((2 * _D, _D)),
        _FULL((1, _D)),
        _FULL((_D, 8 * 128)),
        _FULL((8 * 128, 16)),
        _FULL((8, 8 * 128)),
        _FULL((1, 8 * 128)),
        _FULL((8 * 128, _D)),
        _FULL((_D, 2)),
        _FULL((1, _D)),
        _FULL((4 * _D, 4 * _D)),
        _FULL((1, 4 * _D)),
        _FULL((2, _D)),
        _FULL((16, _D)),
        _FULL((_D, 1)),
        _FULL((_D, 1)),
    ],
    out_specs=[
        pl.BlockSpec((_RB, _SP, 4 * _D), lambda b: (b, 0, 0)),
        pl.BlockSpec((_RB, _SP, 1), lambda b: (b, 0, 0)),
    ],
    out_shape=[
        jax.ShapeDtypeStruct((_B, _SP, 4 * _D), jnp.float32),
        jax.ShapeDtypeStruct((_B, _SP, 1), jnp.float32),
    ],
    compiler_params=pltpu.CompilerParams(dimension_semantics=("arbitrary",)),
)


_UT = 16


def _lstm_body(z_ref, scal_ref, whh_ref, bhh_ref, wh_ref, out_ref, h_ref, c_ref):
    t = pl.program_id(0)

    @pl.when(t == 0)
    def _init():
        h_ref[...] = jnp.zeros((_B, _D), jnp.float32)
        c_ref[...] = jnp.zeros((_B, _D), jnp.float32)

    h = h_ref[...]
    c = c_ref[...]
    sig = lambda v: 1.0 / (1.0 + jnp.exp(-v))
    for u in range(_UT):
        zh = jnp.dot(h, whh_ref[...], preferred_element_type=jnp.float32) \
            + bhh_ref[...]
        zi = z_ref[:, u, 0:_D] + zh[:, 0:_D]
        zf = z_ref[:, u, _D:2 * _D] + zh[:, _D:2 * _D]
        zg = z_ref[:, u, 2 * _D:3 * _D] + zh[:, 2 * _D:3 * _D]
        zo = z_ref[:, u, 3 * _D:] + zh[:, 3 * _D:]
        c = sig(zf) * c + sig(zi) * jnp.tanh(zg)
        h = sig(zo) * jnp.tanh(c)
        hp = jnp.dot(h, wh_ref[...], preferred_element_type=jnp.float32)
        out_ref[u] = hp + scal_ref[u]
    h_ref[...] = h
    c_ref[...] = c
    out_ref[...] = sig(out_ref[...])


_LSTM_KW = dict(
    grid=(_SP // _UT,),
    in_specs=[
        pl.BlockSpec((_B, _UT, 4 * _D), lambda t: (0, t, 0)),
        pl.BlockSpec((_UT, _B, 1), lambda t: (t, 0, 0)),
        _FULL((_D, 4 * _D)),
        _FULL((1, 4 * _D)),
        _FULL((_D, 1)),
    ],
    out_specs=pl.BlockSpec((_UT, _B, 1), lambda t: (t, 0, 0)),
    out_shape=jax.ShapeDtypeStruct((_SP, _B, 1), jnp.float32),
    scratch_shapes=[
        pltpu.VMEM((_B, _D), jnp.float32),
        pltpu.VMEM((_B, _D), jnp.float32),
    ],
    compiler_params=pltpu.CompilerParams(dimension_semantics=("arbitrary",)),
)


def kernel(true_length, p, q, r, aff, q_next, p_next, emb_p, emb_q, emb_r,
           emb_aff, W_aff, b_aff, W1, att_src1, att_dst1, b1, W2, att_src2,
           att_dst2, b2, W_ih, W_hh, b_ih, b_hh, W_out, b_out):
    f32 = jnp.float32
    i32 = jnp.int32
    pad = lambda a: jnp.pad(a.astype(i32), ((0, 0), (0, _SP - _S)))

    idx4 = jnp.stack([pad(p), pad(q), pad(q_next), pad(p_next)]) \
        .reshape(4, _NW, _NCH, _CH)
    pe, qe, qn, pn = [g.reshape(_B, _SP, _D)
                      for g in _sc_gather(emb_p.astype(f32), emb_q.astype(f32),
                                          idx4)]

    aff4 = pad(aff)[..., None]
    r4 = pad(r)[..., None]

    Wa = W_aff.T
    As1 = (att_src1[:, :, None] * jnp.eye(8, dtype=f32)[:, None, :]) \
        .reshape(8 * 128, 8)
    Ad1 = (att_dst1[:, :, None] * jnp.eye(8, dtype=f32)[:, None, :]) \
        .reshape(8 * 128, 8)
    Asd1 = jnp.concatenate([As1, Ad1], axis=1)
    asd2 = jnp.concatenate([att_src2.T, att_dst2.T], axis=1)
    E8 = jnp.kron(jnp.eye(8, dtype=f32), jnp.ones((1, 128), f32))
    embaff16 = jnp.pad(emb_aff, ((0, 5), (0, 0)))
    wqn = W_out[0, _D:2 * _D][:, None]
    wpn = W_out[0, 2 * _D:][:, None]
    wh = W_out[0, :_D][:, None]

    zx, scal3 = pl.pallas_call(_prelude_body, **_PRELUDE_KW)(
        true_length.astype(i32), b_out.astype(f32),
        pe, qe, qn, pn, aff4, r4,
        Wa, b_aff[None], W1.T, Asd1, E8, b1[None],
        W2.T, asd2, b2[None],
        W_ih.T, b_ih[None], emb_r, embaff16, wqn, wpn)

    out_tb = pl.pallas_call(_lstm_body, **_LSTM_KW)(
        zx, jnp.transpose(scal3, (1, 0, 2)), W_hh.T, b_hh[None], wh)

    return jnp.transpose(out_tb[:_S], (1, 0, 2))
